# baseline (speedup 1.0000x reference)
import functools

import jax
import jax.numpy as jnp
from jax import lax
from jax.experimental import pallas as pl
from jax.experimental.pallas import tpu as pltpu
from jax.experimental.pallas import tpu_sc as plsc

VOCAB = 100000
D = 128
B = 16384
NEG = 20
L = 16
NC = 2
NS = 16
NW = NC * NS
PW = B // NW
C = 16
NCHUNK = PW // C
SGRP = 24
SCOL = SGRP * L
NSPLIT = 4
NSUB = C * NEG // NSPLIT
TCBLK = 2048
NBLK = B // TCBLK


def _sc_scores_body(w_hbm, wp_hbm, cen_hbm, ctx_hbm, neg_hbm, out_hbm,
                    ixc, ixo, ixn, vc0, vc1, uo0, uo1, un0, un1,
                    so0, so1, semi, sem0, sem1, semo0, semo1):
    wid = lax.axis_index("s") * NC + lax.axis_index("c")
    base = wid * PW
    vcs, uos, uns = (vc0, vc1), (uo0, uo1), (un0, un1)
    sos, semos = (so0, so1), (semo0, semo1)
    sems = (sem0, sem1)

    pltpu.async_copy(cen_hbm.at[pl.ds(base, PW)], ixc, semi)
    pltpu.async_copy(ctx_hbm.at[pl.ds(base, PW)], ixo, semi)
    pltpu.async_copy(neg_hbm.at[pl.ds(base * NEG, PW * NEG)], ixn, semi)
    pltpu.make_async_copy(cen_hbm.at[pl.ds(0, PW)], ixc, semi).wait()
    pltpu.make_async_copy(ctx_hbm.at[pl.ds(0, PW)], ixo, semi).wait()
    pltpu.make_async_copy(neg_hbm.at[pl.ds(0, PW * NEG)], ixn, semi).wait()

    def issue(g, slot):
        off = g * C
        pltpu.async_copy(w_hbm.at[ixc.at[pl.ds(off, C)]], vcs[slot],
                         sems[slot])
        pltpu.async_copy(wp_hbm.at[ixo.at[pl.ds(off, C)]], uos[slot],
                         sems[slot])
        for j in range(NSPLIT):
            pltpu.async_copy(
                wp_hbm.at[ixn.at[pl.ds(off * NEG + j * NSUB, NSUB)]],
                uns[slot].at[pl.ds(j * NSUB, NSUB)], sems[slot])

    def drain(slot):
        pltpu.make_async_copy(w_hbm.at[pl.ds(0, C)], vcs[slot],
                              sems[slot]).wait()
        pltpu.make_async_copy(wp_hbm.at[pl.ds(0, C)], uos[slot],
                              sems[slot]).wait()
        pltpu.make_async_copy(wp_hbm.at[pl.ds(0, C * NEG)], uns[slot],
                              sems[slot]).wait()

    def drain_out(slot):
        pltpu.make_async_copy(sos[slot], out_hbm.at[pl.ds(0, C)],
                              semos[slot]).wait()

    def compute(g, slot):
        vc, uo, un = vcs[slot], uos[slot], uns[slot]
        so = sos[slot]

        @pl.when(g >= 2)
        def _():
            drain_out(slot)

        def item_body(i, _):
            v = [vc[i, pl.ds(c * L, L)] for c in range(D // L)]
            for k in range(NEG):
                acc = un[i * NEG + k, pl.ds(0, L)] * v[0]
                for c in range(1, D // L):
                    acc = acc + un[i * NEG + k, pl.ds(c * L, L)] * v[c]
                so[i, pl.ds(k * L, L)] = acc
            acc = uo[i, pl.ds(0, L)] * v[0]
            for c in range(1, D // L):
                acc = acc + uo[i, pl.ds(c * L, L)] * v[c]
            so[i, pl.ds(NEG * L, L)] = acc
            zero = jnp.zeros((L,), jnp.float32)
            for k in range(NEG + 1, SGRP):
                so[i, pl.ds(k * L, L)] = zero
            return 0

        lax.fori_loop(0, C, item_body, 0)
        pltpu.async_copy(so, out_hbm.at[pl.ds(base + g * C, C)],
                         semos[slot])

    issue(0, 0)

    def pair_body(h, _):
        g0 = 2 * h
        g1 = g0 + 1
        issue(g1, 1)
        drain(0)
        compute(g0, 0)

        @pl.when(g1 + 1 < NCHUNK)
        def _():
            issue(g1 + 1, 0)

        drain(1)
        compute(g1, 1)
        return 0

    lax.fori_loop(0, NCHUNK // 2, pair_body, 0)
    drain_out(0)
    drain_out(1)


@functools.lru_cache(maxsize=1)
def _sc_scores_fn():
    return pl.kernel(
        _sc_scores_body,
        out_type=jax.ShapeDtypeStruct((B, SCOL), jnp.float32),
        mesh=plsc.VectorSubcoreMesh(core_axis_name="c",
                                    subcore_axis_name="s"),
        scratch_types=[
            pltpu.VMEM((PW,), jnp.int32),
            pltpu.VMEM((PW,), jnp.int32),
            pltpu.VMEM((PW * NEG,), jnp.int32),
            pltpu.VMEM((C, D), jnp.float32),
            pltpu.VMEM((C, D), jnp.float32),
            pltpu.VMEM((C, D), jnp.float32),
            pltpu.VMEM((C, D), jnp.float32),
            pltpu.VMEM((C * NEG, D), jnp.float32),
            pltpu.VMEM((C * NEG, D), jnp.float32),
            pltpu.VMEM((C, SCOL), jnp.float32),
            pltpu.VMEM((C, SCOL), jnp.float32),
            pltpu.SemaphoreType.DMA,
            pltpu.SemaphoreType.DMA,
            pltpu.SemaphoreType.DMA,
            pltpu.SemaphoreType.DMA,
            pltpu.SemaphoreType.DMA,
        ],
    )


def _tc_loss_body(s_ref, o_ref, acc_ref):
    j = pl.program_id(0)

    @pl.when(j == 0)
    def _():
        acc_ref[0] = 0.0

    x = s_ref[...].astype(jnp.bfloat16)
    r = lax.broadcasted_iota(jnp.int32, (SCOL, 128), 0)
    c = lax.broadcasted_iota(jnp.int32, (SCOL, 128), 1)
    m = (r // L == c).astype(jnp.bfloat16)
    s = jnp.dot(x, m, preferred_element_type=jnp.float32)
    col = lax.broadcasted_iota(jnp.int32, s.shape, 1)
    t = jnp.where(col == NEG, s, -s)
    term = jnp.where(col <= NEG, jnp.log(jax.nn.sigmoid(t) + 1e-10), 0.0)
    acc_ref[0] += jnp.sum(term)

    @pl.when(j == NBLK - 1)
    def _():
        o_ref[0, 0] = -acc_ref[0] / B


def _tc_loss(scores):
    out = pl.pallas_call(
        _tc_loss_body,
        grid=(NBLK,),
        in_specs=[pl.BlockSpec((TCBLK, SCOL), lambda j: (j, 0))],
        out_specs=pl.BlockSpec(memory_space=pltpu.SMEM),
        out_shape=jax.ShapeDtypeStruct((1, 1), jnp.float32),
        scratch_shapes=[pltpu.SMEM((1,), jnp.float32)],
    )(scores)
    return out[0, 0]


def kernel(W, W_prime, centers, contexts, negatives):
    cen = centers.astype(jnp.int32)
    ctx = contexts.astype(jnp.int32)
    neg = negatives.astype(jnp.int32).reshape(-1)
    scores = _sc_scores_fn()(W, W_prime, cen, ctx, neg)
    return _tc_loss(scores)


# speedup vs baseline: 1.4858x; 1.4858x over previous
import functools

import jax
import jax.numpy as jnp
from jax import lax
from jax.experimental import pallas as pl
from jax.experimental.pallas import tpu as pltpu
from jax.experimental.pallas import tpu_sc as plsc

VOCAB = 100000
D = 128
B = 16384
NEG = 20
L = 16
NC = 2
NS = 16
NW = NC * NS
PW = B // NW
C = 16
NCHUNK = PW // C
SGRP = 24
SCOL = SGRP * L
NSPLIT = 4
NSUB = C * NEG // NSPLIT
TCBLK = 2048
NBLK = B // TCBLK


def _sc_scores_body(w_hbm, wp_hbm, cen_hbm, ctx_hbm, neg_hbm, out_hbm,
                    ixc, ixo, ixn, vc0, vc1, uo0, uo1, un0, un1,
                    so0, so1, semi, sem0, sem1, semo0, semo1):
    wid = lax.axis_index("s") * NC + lax.axis_index("c")
    base = wid * PW
    vcs, uos, uns = (vc0, vc1), (uo0, uo1), (un0, un1)
    sos, semos = (so0, so1), (semo0, semo1)
    sems = (sem0, sem1)

    pltpu.async_copy(cen_hbm.at[pl.ds(base, PW)], ixc, semi)
    pltpu.async_copy(ctx_hbm.at[pl.ds(base, PW)], ixo, semi)
    pltpu.async_copy(neg_hbm.at[pl.ds(base * NEG, PW * NEG)], ixn, semi)
    pltpu.make_async_copy(cen_hbm.at[pl.ds(0, PW)], ixc, semi).wait()
    pltpu.make_async_copy(ctx_hbm.at[pl.ds(0, PW)], ixo, semi).wait()
    pltpu.make_async_copy(neg_hbm.at[pl.ds(0, PW * NEG)], ixn, semi).wait()

    def issue(g, slot):
        off = g * C
        pltpu.async_copy(w_hbm.at[ixc.at[pl.ds(off, C)]], vcs[slot],
                         sems[slot])
        pltpu.async_copy(wp_hbm.at[ixo.at[pl.ds(off, C)]], uos[slot],
                         sems[slot])
        for j in range(NSPLIT):
            pltpu.async_copy(
                wp_hbm.at[ixn.at[pl.ds(off * NEG + j * NSUB, NSUB)]],
                uns[slot].at[pl.ds(j * NSUB, NSUB)], sems[slot])

    def drain(slot):
        pltpu.make_async_copy(w_hbm.at[pl.ds(0, C)], vcs[slot],
                              sems[slot]).wait()
        pltpu.make_async_copy(wp_hbm.at[pl.ds(0, C)], uos[slot],
                              sems[slot]).wait()
        pltpu.make_async_copy(wp_hbm.at[pl.ds(0, C * NEG)], uns[slot],
                              sems[slot]).wait()

    def drain_out(slot):
        pltpu.make_async_copy(sos[slot], out_hbm.at[pl.ds(0, C)],
                              semos[slot]).wait()

    def compute(g, slot):
        vc, uo, un = vcs[slot], uos[slot], uns[slot]
        so = sos[slot]

        @pl.when(g >= 2)
        def _():
            drain_out(slot)

        def item_body(i, _):
            v = [vc[i, pl.ds(c * L, L)] for c in range(D // L)]
            zero = jnp.zeros((L,), jnp.float32)
            accs = []
            for k in range(NEG):
                acc = un[i * NEG + k, pl.ds(0, L)] * v[0]
                for c in range(1, D // L):
                    acc = acc + un[i * NEG + k, pl.ds(c * L, L)] * v[c]
                accs.append(acc)
            acc = uo[i, pl.ds(0, L)] * v[0]
            for c in range(1, D // L):
                acc = acc + uo[i, pl.ds(c * L, L)] * v[c]
            accs.append(acc)
            while len(accs) < SGRP:
                accs.append(zero)
            for k in range(SGRP):
                so[i, pl.ds(k * L, L)] = accs[k]
            return 0

        lax.fori_loop(0, C, item_body, 0)
        pltpu.async_copy(so, out_hbm.at[pl.ds(base + g * C, C)],
                         semos[slot])

    issue(0, 0)

    def pair_body(h, _):
        g0 = 2 * h
        g1 = g0 + 1
        issue(g1, 1)
        drain(0)
        compute(g0, 0)

        @pl.when(g1 + 1 < NCHUNK)
        def _():
            issue(g1 + 1, 0)

        drain(1)
        compute(g1, 1)
        return 0

    lax.fori_loop(0, NCHUNK // 2, pair_body, 0)
    drain_out(0)
    drain_out(1)


@functools.lru_cache(maxsize=1)
def _sc_scores_fn():
    return pl.kernel(
        _sc_scores_body,
        out_type=jax.ShapeDtypeStruct((B, SCOL), jnp.float32),
        mesh=plsc.VectorSubcoreMesh(core_axis_name="c",
                                    subcore_axis_name="s"),
        scratch_types=[
            pltpu.VMEM((PW,), jnp.int32),
            pltpu.VMEM((PW,), jnp.int32),
            pltpu.VMEM((PW * NEG,), jnp.int32),
            pltpu.VMEM((C, D), jnp.float32),
            pltpu.VMEM((C, D), jnp.float32),
            pltpu.VMEM((C, D), jnp.float32),
            pltpu.VMEM((C, D), jnp.float32),
            pltpu.VMEM((C * NEG, D), jnp.float32),
            pltpu.VMEM((C * NEG, D), jnp.float32),
            pltpu.VMEM((C, SCOL), jnp.float32),
            pltpu.VMEM((C, SCOL), jnp.float32),
            pltpu.SemaphoreType.DMA,
            pltpu.SemaphoreType.DMA,
            pltpu.SemaphoreType.DMA,
            pltpu.SemaphoreType.DMA,
            pltpu.SemaphoreType.DMA,
        ],
    )


def _tc_loss_body(s_ref, o_ref, acc_ref):
    j = pl.program_id(0)

    @pl.when(j == 0)
    def _():
        acc_ref[0] = 0.0

    x = s_ref[...].astype(jnp.bfloat16)
    r = lax.broadcasted_iota(jnp.int32, (SCOL, 128), 0)
    c = lax.broadcasted_iota(jnp.int32, (SCOL, 128), 1)
    m = (r // L == c).astype(jnp.bfloat16)
    s = jnp.dot(x, m, preferred_element_type=jnp.float32)
    col = lax.broadcasted_iota(jnp.int32, s.shape, 1)
    t = jnp.where(col == NEG, s, -s)
    term = jnp.where(col <= NEG, jnp.log(jax.nn.sigmoid(t) + 1e-10), 0.0)
    acc_ref[0] += jnp.sum(term)

    @pl.when(j == NBLK - 1)
    def _():
        o_ref[0, 0] = -acc_ref[0] / B


def _tc_loss(scores):
    out = pl.pallas_call(
        _tc_loss_body,
        grid=(NBLK,),
        in_specs=[pl.BlockSpec((TCBLK, SCOL), lambda j: (j, 0))],
        out_specs=pl.BlockSpec(memory_space=pltpu.SMEM),
        out_shape=jax.ShapeDtypeStruct((1, 1), jnp.float32),
        scratch_shapes=[pltpu.SMEM((1,), jnp.float32)],
    )(scores)
    return out[0, 0]


def kernel(W, W_prime, centers, contexts, negatives):
    cen = centers.astype(jnp.int32)
    ctx = contexts.astype(jnp.int32)
    neg = negatives.astype(jnp.int32).reshape(-1)
    scores = _sc_scores_fn()(W, W_prime, cen, ctx, neg)
    return _tc_loss(scores)


# speedup vs baseline: 1.5061x; 1.0136x over previous
import functools

import jax
import jax.numpy as jnp
from jax import lax
from jax.experimental import pallas as pl
from jax.experimental.pallas import tpu as pltpu
from jax.experimental.pallas import tpu_sc as plsc

VOCAB = 100000
D = 128
B = 16384
NEG = 20
L = 16
NC = 2
NS = 16
NW = NC * NS
PW = B // NW
C = 16
NCHUNK = PW // C
SGRP = 24
SCOL = SGRP * L
NSPLIT = 4
NSUB = C * NEG // NSPLIT
TCBLK = 4096
NBLK = B // TCBLK


def _sc_scores_body(w_hbm, wp_hbm, cen_hbm, ctx_hbm, neg_hbm, out_hbm,
                    ixc, ixo, ixn, vc0, vc1, uo0, uo1, un0, un1,
                    so0, so1, semi, sem0, sem1, semo0, semo1):
    wid = lax.axis_index("s") * NC + lax.axis_index("c")
    base = wid * PW
    vcs, uos, uns = (vc0, vc1), (uo0, uo1), (un0, un1)
    sos, semos = (so0, so1), (semo0, semo1)
    sems = (sem0, sem1)

    pltpu.async_copy(cen_hbm.at[pl.ds(base, PW)], ixc, semi)
    pltpu.async_copy(ctx_hbm.at[pl.ds(base, PW)], ixo, semi)
    pltpu.async_copy(neg_hbm.at[pl.ds(base * NEG, PW * NEG)], ixn, semi)
    pltpu.make_async_copy(cen_hbm.at[pl.ds(0, PW)], ixc, semi).wait()
    pltpu.make_async_copy(ctx_hbm.at[pl.ds(0, PW)], ixo, semi).wait()
    pltpu.make_async_copy(neg_hbm.at[pl.ds(0, PW * NEG)], ixn, semi).wait()

    def issue(g, slot):
        off = g * C
        pltpu.async_copy(w_hbm.at[ixc.at[pl.ds(off, C)]], vcs[slot],
                         sems[slot])
        pltpu.async_copy(wp_hbm.at[ixo.at[pl.ds(off, C)]], uos[slot],
                         sems[slot])
        for j in range(NSPLIT):
            pltpu.async_copy(
                wp_hbm.at[ixn.at[pl.ds(off * NEG + j * NSUB, NSUB)]],
                uns[slot].at[pl.ds(j * NSUB, NSUB)], sems[slot])

    def drain(slot):
        pltpu.make_async_copy(w_hbm.at[pl.ds(0, C)], vcs[slot],
                              sems[slot]).wait()
        pltpu.make_async_copy(wp_hbm.at[pl.ds(0, C)], uos[slot],
                              sems[slot]).wait()
        pltpu.make_async_copy(wp_hbm.at[pl.ds(0, C * NEG)], uns[slot],
                              sems[slot]).wait()

    def drain_out(slot):
        pltpu.make_async_copy(sos[slot], out_hbm.at[pl.ds(0, C)],
                              semos[slot]).wait()

    def compute(g, slot):
        vc, uo, un = vcs[slot], uos[slot], uns[slot]
        so = sos[slot]

        @pl.when(g >= 2)
        def _():
            drain_out(slot)

        def item_body(i, _):
            v = [vc[i, pl.ds(c * L, L)] for c in range(D // L)]
            zero = jnp.zeros((L,), jnp.float32)
            accs = []
            for k in range(NEG):
                acc = un[i * NEG + k, pl.ds(0, L)] * v[0]
                for c in range(1, D // L):
                    acc = acc + un[i * NEG + k, pl.ds(c * L, L)] * v[c]
                accs.append(acc)
            acc = uo[i, pl.ds(0, L)] * v[0]
            for c in range(1, D // L):
                acc = acc + uo[i, pl.ds(c * L, L)] * v[c]
            accs.append(acc)
            while len(accs) < SGRP:
                accs.append(zero)
            for k in range(SGRP):
                so[i, pl.ds(k * L, L)] = accs[k]
            return 0

        lax.fori_loop(0, C, item_body, 0)
        pltpu.async_copy(so, out_hbm.at[pl.ds(base + g * C, C)],
                         semos[slot])

    issue(0, 0)

    def pair_body(h, _):
        g0 = 2 * h
        g1 = g0 + 1
        issue(g1, 1)
        drain(0)
        compute(g0, 0)

        @pl.when(g1 + 1 < NCHUNK)
        def _():
            issue(g1 + 1, 0)

        drain(1)
        compute(g1, 1)
        return 0

    lax.fori_loop(0, NCHUNK // 2, pair_body, 0)
    drain_out(0)
    drain_out(1)


@functools.lru_cache(maxsize=1)
def _sc_scores_fn():
    return pl.kernel(
        _sc_scores_body,
        out_type=jax.ShapeDtypeStruct((B, SCOL), jnp.float32),
        mesh=plsc.VectorSubcoreMesh(core_axis_name="c",
                                    subcore_axis_name="s"),
        scratch_types=[
            pltpu.VMEM((PW,), jnp.int32),
            pltpu.VMEM((PW,), jnp.int32),
            pltpu.VMEM((PW * NEG,), jnp.int32),
            pltpu.VMEM((C, D), jnp.float32),
            pltpu.VMEM((C, D), jnp.float32),
            pltpu.VMEM((C, D), jnp.float32),
            pltpu.VMEM((C, D), jnp.float32),
            pltpu.VMEM((C * NEG, D), jnp.float32),
            pltpu.VMEM((C * NEG, D), jnp.float32),
            pltpu.VMEM((C, SCOL), jnp.float32),
            pltpu.VMEM((C, SCOL), jnp.float32),
            pltpu.SemaphoreType.DMA,
            pltpu.SemaphoreType.DMA,
            pltpu.SemaphoreType.DMA,
            pltpu.SemaphoreType.DMA,
            pltpu.SemaphoreType.DMA,
        ],
    )


def _tc_loss_body(s_ref, o_ref, acc_ref):
    j = pl.program_id(0)

    @pl.when(j == 0)
    def _():
        acc_ref[0] = 0.0

    x = s_ref[...].astype(jnp.bfloat16)
    r = lax.broadcasted_iota(jnp.int32, (SCOL, 128), 0)
    c = lax.broadcasted_iota(jnp.int32, (SCOL, 128), 1)
    m = (r // L == c).astype(jnp.bfloat16)
    s = jnp.dot(x, m, preferred_element_type=jnp.float32)
    col = lax.broadcasted_iota(jnp.int32, s.shape, 1)
    t = jnp.where(col == NEG, s, -s)
    term = jnp.where(col <= NEG, jnp.log(jax.nn.sigmoid(t) + 1e-10), 0.0)
    acc_ref[0] += jnp.sum(term)

    @pl.when(j == NBLK - 1)
    def _():
        o_ref[0, 0] = -acc_ref[0] / B


def _tc_loss(scores):
    out = pl.pallas_call(
        _tc_loss_body,
        grid=(NBLK,),
        in_specs=[pl.BlockSpec((TCBLK, SCOL), lambda j: (j, 0))],
        out_specs=pl.BlockSpec(memory_space=pltpu.SMEM),
        out_shape=jax.ShapeDtypeStruct((1, 1), jnp.float32),
        scratch_shapes=[pltpu.SMEM((1,), jnp.float32)],
    )(scores)
    return out[0, 0]


def kernel(W, W_prime, centers, contexts, negatives):
    cen = centers.astype(jnp.int32)
    ctx = contexts.astype(jnp.int32)
    neg = negatives.astype(jnp.int32).reshape(-1)
    scores = _sc_scores_fn()(W, W_prime, cen, ctx, neg)
    return _tc_loss(scores)
